# baseline (speedup 1.0000x reference)
import functools

import jax
import jax.numpy as jnp
from jax.experimental import pallas as pl
from jax.experimental.pallas import tpu as pltpu

_ALPHA = 0.2
_NEG = -9e15


def _dot(a, b):
    return jnp.dot(a, b, preferred_element_type=jnp.float32)


def _dott(a, b):
    return jax.lax.dot_general(a, b, (((0,), (0,)), ((), ())),
                               preferred_element_type=jnp.float32)


def _lrelu(x):
    return jnp.where(x > 0, x, _ALPHA * x)


def _edge1_kernel(x_ref, h_ref, w1_ref, w2_ref, w3_ref, a_ref, a2_ref, q_ref,
                  edge_ref, te_ref, tn_ref, ht8_ref, wxt_scr, w_scr, *, d):
    i = pl.program_id(0)

    @pl.when(i == 0)
    def _():
        xb = x_ref[...].astype(jnp.bfloat16)
        x4 = _dot(xb, w2_ref[...].astype(jnp.bfloat16))
        xt = _dot(xb, w1_ref[...].astype(jnp.bfloat16))
        sq = _dot(q_ref[...], a_ref[:d, :])
        ln = _lrelu(_dot(x4, a_ref[d:, :]) + sq[0, 0])
        m = jnp.max(ln)
        w = jnp.exp(ln - m)
        w_scr[...] = w.astype(jnp.bfloat16)
        wxt_scr[...] = (xt * w).astype(jnp.bfloat16)
        tn_ref[...] = _dot(x4, a2_ref[:d, :])

    hb = h_ref[...]
    ht8_ref[...] = hb.T.astype(jnp.int8)
    hb_bf = hb.astype(jnp.bfloat16)
    num = _dot(hb_bf, wxt_scr[...])
    z = _dot(hb_bf, w_scr[...])
    edge = num / z
    edge_ref[...] = edge.astype(jnp.bfloat16)
    e4 = _dot(edge.astype(jnp.bfloat16),
              w3_ref[...].astype(jnp.bfloat16))
    te_ref[...] = jax.lax.dot_general(
        a2_ref[d:, :], e4, (((0,), (1,)), ((), ())),
        preferred_element_type=jnp.float32)


def _edge1(x2, h2, w1, w2, w3, a, a2, q, eb):
    e, n = h2.shape
    d = x2.shape[1]
    grid = pl.cdiv(e, eb)
    e_pad = grid * eb
    return pl.pallas_call(
        functools.partial(_edge1_kernel, d=d),
        grid=(grid,),
        in_specs=[
            pl.BlockSpec((n, d), lambda i: (0, 0)),
            pl.BlockSpec((eb, n), lambda i: (i, 0)),
            pl.BlockSpec((d, d), lambda i: (0, 0)),
            pl.BlockSpec((d, d), lambda i: (0, 0)),
            pl.BlockSpec((d, d), lambda i: (0, 0)),
            pl.BlockSpec((2 * d, 1), lambda i: (0, 0)),
            pl.BlockSpec((2 * d, 1), lambda i: (0, 0)),
            pl.BlockSpec((1, d), lambda i: (0, 0)),
        ],
        out_specs=[
            pl.BlockSpec((eb, d), lambda i: (i, 0)),
            pl.BlockSpec((1, eb), lambda i: (0, i)),
            pl.BlockSpec((n, 1), lambda i: (0, 0)),
            pl.BlockSpec((n, eb), lambda i: (0, i)),
        ],
        out_shape=[
            jax.ShapeDtypeStruct((e, d), jnp.bfloat16),
            jax.ShapeDtypeStruct((1, e_pad), jnp.float32),
            jax.ShapeDtypeStruct((n, 1), jnp.float32),
            jax.ShapeDtypeStruct((n, e_pad), jnp.int8),
        ],
        scratch_shapes=[
            pltpu.VMEM((n, d), jnp.bfloat16),
            pltpu.VMEM((n, 1), jnp.bfloat16),
        ],
    )(x2, h2, w1, w2, w3, a, a2, q)


def _node_block(ht, te_ref, tn_ref, edge_ref, xin_ref, e):
    te = te_ref[:, :e].astype(jnp.bfloat16)
    tn = tn_ref[...].astype(jnp.bfloat16)
    s = te + tn
    p = jnp.exp(jnp.maximum(s, jnp.bfloat16(_ALPHA) * s)) * ht[:, :e]
    d = edge_ref.shape[1]
    aug = jnp.concatenate(
        [edge_ref[...], jnp.ones((e, 1), jnp.bfloat16)], axis=1)
    na = _dot(p, aug)
    node = na[:, :d] * (1.0 / na[:, d:])
    elu = jnp.where(node > 0, node, jnp.exp(jnp.minimum(node, 0.0)) - 1.0)
    return elu + xin_ref[...]


def _mid_kernel(ht8_ref, te_ref, tn_ref, edge_ref, xin_ref,
                w1_ref, w2_ref, w3_ref, a_ref, a2_ref, q_ref,
                h_ref, tn2_ref, edge2_ref, te2_ref,
                numt_scr, z_scr, m_scr, *, e, d, nsteps):
    i = pl.program_id(0)
    htf = ht8_ref[...].astype(jnp.bfloat16)
    h = _node_block(htf, te_ref, tn_ref, edge_ref, xin_ref, e)
    h_bf = h.astype(jnp.bfloat16)
    h_ref[...] = h_bf

    x4 = _dot(h_bf, w2_ref[...].astype(jnp.bfloat16))
    xt = _dot(h_bf, w1_ref[...].astype(jnp.bfloat16))
    sq = _dot(q_ref[...], a_ref[:d, :])
    ln = _lrelu(_dot(x4, a_ref[d:, :]) + sq[0, 0])
    tn2_ref[...] = _dot(x4, a2_ref[:d, :])
    mj = jnp.max(ln)

    @pl.when(i == 0)
    def _():
        m_scr[0, 0] = mj
        w = jnp.exp(ln - mj)
        numt_scr[...] = _dott((xt * w).astype(jnp.bfloat16), htf)
        z_scr[...] = _dott(w.astype(jnp.bfloat16), htf)

    @pl.when(i > 0)
    def _():
        m_old = m_scr[0, 0]
        m_new = jnp.maximum(m_old, mj)
        m_scr[0, 0] = m_new
        sc = jnp.exp(m_old - m_new)
        w = jnp.exp(ln - m_new)
        numt_scr[...] = numt_scr[...] * sc + _dott((xt * w).astype(jnp.bfloat16), htf)
        z_scr[...] = z_scr[...] * sc + _dott(w.astype(jnp.bfloat16), htf)

    @pl.when(i == nsteps - 1)
    def _():
        edge2t = numt_scr[...] / z_scr[...]
        edge2_ref[...] = edge2t.T[:e, :].astype(jnp.bfloat16)
        e4t = _dott(w3_ref[...], edge2t)
        te2_ref[...] = _dott(a2_ref[d:, :], e4t)


def _mid(ht8, te, tn, edge, xin, w1, w2, w3, a, a2, q, nb):
    n, e_pad = ht8.shape
    e, d = edge.shape
    grid = n // nb
    return pl.pallas_call(
        functools.partial(_mid_kernel, e=e, d=d, nsteps=grid),
        grid=(grid,),
        in_specs=[
            pl.BlockSpec((nb, e_pad), lambda i: (i, 0)),
            pl.BlockSpec((1, e_pad), lambda i: (0, 0)),
            pl.BlockSpec((nb, 1), lambda i: (i, 0)),
            pl.BlockSpec((e, d), lambda i: (0, 0)),
            pl.BlockSpec((nb, d), lambda i: (i, 0)),
            pl.BlockSpec((d, d), lambda i: (0, 0)),
            pl.BlockSpec((d, d), lambda i: (0, 0)),
            pl.BlockSpec((d, d), lambda i: (0, 0)),
            pl.BlockSpec((2 * d, 1), lambda i: (0, 0)),
            pl.BlockSpec((2 * d, 1), lambda i: (0, 0)),
            pl.BlockSpec((1, d), lambda i: (0, 0)),
        ],
        out_specs=[
            pl.BlockSpec((nb, d), lambda i: (i, 0)),
            pl.BlockSpec((nb, 1), lambda i: (i, 0)),
            pl.BlockSpec((e, d), lambda i: (0, 0)),
            pl.BlockSpec((1, e_pad), lambda i: (0, 0)),
        ],
        out_shape=[
            jax.ShapeDtypeStruct((n, d), jnp.bfloat16),
            jax.ShapeDtypeStruct((n, 1), jnp.float32),
            jax.ShapeDtypeStruct((e, d), jnp.bfloat16),
            jax.ShapeDtypeStruct((1, e_pad), jnp.float32),
        ],
        scratch_shapes=[
            pltpu.VMEM((d, e_pad), jnp.float32),
            pltpu.VMEM((1, e_pad), jnp.float32),
            pltpu.SMEM((1, 1), jnp.float32),
        ],
    )(ht8, te, tn, edge, xin, w1, w2, w3, a, a2, q)


def _node_kernel(ht8_ref, te_ref, tn_ref, edge_ref, xin_ref, out_ref, *, e):
    htf = ht8_ref[...].astype(jnp.bfloat16)
    out_ref[...] = _node_block(htf, te_ref, tn_ref, edge_ref, xin_ref, e)


def _node(ht8, te, tn, edge, xin, nb):
    n, e_pad = ht8.shape
    e, d = edge.shape
    grid = n // nb
    return pl.pallas_call(
        functools.partial(_node_kernel, e=e),
        grid=(grid,),
        in_specs=[
            pl.BlockSpec((nb, e_pad), lambda i: (i, 0)),
            pl.BlockSpec((1, e_pad), lambda i: (0, 0)),
            pl.BlockSpec((nb, 1), lambda i: (i, 0)),
            pl.BlockSpec((e, d), lambda i: (0, 0)),
            pl.BlockSpec((nb, d), lambda i: (i, 0)),
        ],
        out_specs=pl.BlockSpec((nb, d), lambda i: (i, 0)),
        out_shape=jax.ShapeDtypeStruct((n, d), jnp.float32),
    )(ht8, te, tn, edge, xin)


def kernel(x, H, W1_1, W2_1, W3_1, a_1, a2_1, q_1,
           W1_2, W2_2, W3_2, a_2, a2_2, q_2):
    x2 = x[0]
    h2 = H[0]
    eb = 256
    nb_node = 2000

    edge1, te1, tn1, ht8 = _edge1(
        x2, h2, W1_1, W2_1, W3_1, a_1, a2_1, q_1, eb)
    h, tn2, edge2, te2 = _mid(
        ht8, te1, tn1, edge1, x2, W1_2, W2_2, W3_2, a_2, a2_2, q_2, nb_node)
    out = _node(ht8, te2, tn2, edge2, h, nb_node)
    return out[None]


# speedup vs baseline: 1.0524x; 1.0524x over previous
import functools

import jax
import jax.numpy as jnp
from jax.experimental import pallas as pl
from jax.experimental.pallas import tpu as pltpu

_ALPHA = 0.2
_NEG = -9e15


def _dot(a, b):
    return jnp.dot(a, b, preferred_element_type=jnp.float32)


def _dott(a, b):
    return jax.lax.dot_general(a, b, (((0,), (0,)), ((), ())),
                               preferred_element_type=jnp.float32)


def _lrelu(x):
    return jnp.where(x > 0, x, _ALPHA * x)


def _edge1_kernel(x_ref, h_ref, w1_ref, w2_ref, w3_ref, a_ref, a2_ref, q_ref,
                  edge_ref, te_ref, tn_ref, ht8_ref, wxt_scr, w_scr, *, d):
    i = pl.program_id(0)

    @pl.when(i == 0)
    def _():
        xb = x_ref[...]
        x4 = _dot(xb, w2_ref[...])
        xt = _dot(xb, w1_ref[...])
        sq = _dot(q_ref[...], a_ref[:d, :])
        ln = _lrelu(_dot(x4, a_ref[d:, :]) + sq[0, 0])
        m = jnp.max(ln)
        w = jnp.exp(ln - m)
        w_scr[...] = w.astype(jnp.bfloat16)
        wxt_scr[...] = (xt * w).astype(jnp.bfloat16)
        tn_ref[...] = _dot(x4, a2_ref[:d, :])

    hb = h_ref[...]
    ht8_ref[...] = hb.T.astype(jnp.int8)
    hb_bf = hb.astype(jnp.bfloat16)
    num = _dot(hb_bf, wxt_scr[...])
    z = _dot(hb_bf, w_scr[...])
    edge = num / z
    edge_ref[...] = edge.astype(jnp.bfloat16)
    e4 = _dot(edge, w3_ref[...])
    te_ref[...] = jax.lax.dot_general(
        a2_ref[d:, :], e4, (((0,), (1,)), ((), ())),
        preferred_element_type=jnp.float32)


def _edge1(x2, h2, w1, w2, w3, a, a2, q, eb):
    e, n = h2.shape
    d = x2.shape[1]
    grid = pl.cdiv(e, eb)
    e_pad = grid * eb
    return pl.pallas_call(
        functools.partial(_edge1_kernel, d=d),
        grid=(grid,),
        in_specs=[
            pl.BlockSpec((n, d), lambda i: (0, 0)),
            pl.BlockSpec((eb, n), lambda i: (i, 0)),
            pl.BlockSpec((d, d), lambda i: (0, 0)),
            pl.BlockSpec((d, d), lambda i: (0, 0)),
            pl.BlockSpec((d, d), lambda i: (0, 0)),
            pl.BlockSpec((2 * d, 1), lambda i: (0, 0)),
            pl.BlockSpec((2 * d, 1), lambda i: (0, 0)),
            pl.BlockSpec((1, d), lambda i: (0, 0)),
        ],
        out_specs=[
            pl.BlockSpec((eb, d), lambda i: (i, 0)),
            pl.BlockSpec((1, eb), lambda i: (0, i)),
            pl.BlockSpec((n, 1), lambda i: (0, 0)),
            pl.BlockSpec((n, eb), lambda i: (0, i)),
        ],
        out_shape=[
            jax.ShapeDtypeStruct((e, d), jnp.bfloat16),
            jax.ShapeDtypeStruct((1, e_pad), jnp.float32),
            jax.ShapeDtypeStruct((n, 1), jnp.float32),
            jax.ShapeDtypeStruct((n, e_pad), jnp.int8),
        ],
        scratch_shapes=[
            pltpu.VMEM((n, d), jnp.bfloat16),
            pltpu.VMEM((n, 1), jnp.bfloat16),
        ],
    )(x2, h2, w1, w2, w3, a, a2, q)


def _node_attn(ht, te_row, tn_col, edge, e):
    d = edge.shape[1]
    te = te_row[:, :e].astype(jnp.bfloat16)
    tn = tn_col.astype(jnp.bfloat16)
    s = te + tn
    p = jnp.exp(jnp.maximum(s, jnp.bfloat16(_ALPHA) * s)) * ht[:, :e]
    aug = jnp.concatenate(
        [edge, jnp.ones((e, 1), jnp.bfloat16)], axis=1)
    na = _dot(p, aug)
    node = na[:, :d] * (1.0 / na[:, d:])
    return jnp.where(node > 0, node, jnp.exp(jnp.minimum(node, 0.0)) - 1.0)


def _mid_kernel(ht8_ref, te_ref, tn_ref, edge_ref, xin_ref,
                w1_ref, w2_ref, w3_ref, a_ref, a2_ref, q_ref,
                out_ref,
                h_scr, tn2_scr, edge2_scr, te2_scr,
                numt_scr, z_scr, m_scr, *, e, d, nb, nsteps):
    i = pl.program_id(0)
    htf = ht8_ref[...].astype(jnp.bfloat16)

    @pl.when(i < nsteps)
    def _():
        h = _node_attn(htf, te_ref[...], tn_ref[...], edge_ref[...], e)
        h = h + xin_ref[...]
        row = pl.ds((i % nsteps) * nb, nb)
        h_scr[row, :] = h.astype(jnp.bfloat16)

        x4 = _dot(h, w2_ref[...])
        xt = _dot(h, w1_ref[...])
        sq = _dot(q_ref[...], a_ref[:d, :])
        ln = _lrelu(_dot(x4, a_ref[d:, :]) + sq[0, 0])
        tn2_scr[row, :] = _dot(x4, a2_ref[:d, :])
        mj = jnp.max(ln)

        @pl.when(i == 0)
        def _():
            m_scr[0, 0] = mj
            w = jnp.exp(ln - mj)
            numt_scr[...] = _dott((xt * w).astype(jnp.bfloat16), htf)
            z_scr[...] = _dott(w.astype(jnp.bfloat16), htf)

        @pl.when(i > 0)
        def _():
            m_old = m_scr[0, 0]
            m_new = jnp.maximum(m_old, mj)
            m_scr[0, 0] = m_new
            sc = jnp.exp(m_old - m_new)
            w = jnp.exp(ln - m_new)
            numt_scr[...] = numt_scr[...] * sc + _dott(
                (xt * w).astype(jnp.bfloat16), htf)
            z_scr[...] = z_scr[...] * sc + _dott(w.astype(jnp.bfloat16), htf)

        @pl.when(i == nsteps - 1)
        def _():
            edge2t = numt_scr[...] / z_scr[...]
            edge2_scr[...] = edge2t.T[:e, :].astype(jnp.bfloat16)
            e4t = _dott(w3_ref[...], edge2t)
            te2_scr[...] = _dott(a2_ref[d:, :], e4t)

    @pl.when(i >= nsteps)
    def _():
        row = pl.ds((i % nsteps) * nb, nb)
        node = _node_attn(htf, te2_scr[...], tn2_scr[row, :],
                          edge2_scr[...], e)
        out_ref[...] = node + h_scr[row, :]


def _mid(ht8, te, tn, edge, xin, w1, w2, w3, a, a2, q, nb):
    n, e_pad = ht8.shape
    e, d = edge.shape
    g = n // nb
    return pl.pallas_call(
        functools.partial(_mid_kernel, e=e, d=d, nb=nb, nsteps=g),
        grid=(2 * g,),
        in_specs=[
            pl.BlockSpec((nb, e_pad), lambda i: (i % g, 0)),
            pl.BlockSpec((1, e_pad), lambda i: (0, 0)),
            pl.BlockSpec((nb, 1), lambda i: (i % g, 0)),
            pl.BlockSpec((e, d), lambda i: (0, 0)),
            pl.BlockSpec((nb, d), lambda i: (jnp.where(i < g, i, g - 1), 0)),
            pl.BlockSpec((d, d), lambda i: (0, 0)),
            pl.BlockSpec((d, d), lambda i: (0, 0)),
            pl.BlockSpec((d, d), lambda i: (0, 0)),
            pl.BlockSpec((2 * d, 1), lambda i: (0, 0)),
            pl.BlockSpec((2 * d, 1), lambda i: (0, 0)),
            pl.BlockSpec((1, d), lambda i: (0, 0)),
        ],
        out_specs=pl.BlockSpec((nb, d), lambda i: (jnp.where(i < g, 0, i - g), 0)),
        out_shape=jax.ShapeDtypeStruct((n, d), jnp.float32),
        scratch_shapes=[
            pltpu.VMEM((n, d), jnp.bfloat16),
            pltpu.VMEM((n, 1), jnp.float32),
            pltpu.VMEM((e, d), jnp.bfloat16),
            pltpu.VMEM((1, e_pad), jnp.float32),
            pltpu.VMEM((d, e_pad), jnp.float32),
            pltpu.VMEM((1, e_pad), jnp.float32),
            pltpu.SMEM((1, 1), jnp.float32),
        ],
    )(ht8, te, tn, edge, xin, w1, w2, w3, a, a2, q)


def kernel(x, H, W1_1, W2_1, W3_1, a_1, a2_1, q_1,
           W1_2, W2_2, W3_2, a_2, a2_2, q_2):
    x2 = x[0]
    h2 = H[0]
    eb = 256
    nb_node = 2000

    edge1, te1, tn1, ht8 = _edge1(
        x2, h2, W1_1, W2_1, W3_1, a_1, a2_1, q_1, eb)
    out = _mid(
        ht8, te1, tn1, edge1, x2, W1_2, W2_2, W3_2, a_2, a2_2, q_2, nb_node)
    return out[None]
